# Initial kernel scaffold; baseline (speedup 1.0000x reference)
import jax
import jax.numpy as jnp
from jax.experimental import pallas as pl


def kernel(hD, hE, edge_d2e, edge_e2d, error_weights, W_d2e, W_e_self, b_e, g_e, beta_e, W_e2d, W_d_self, b_d, g_d, beta_d):
    raise NotImplementedError("write your pallas kernel here")


# speedup vs baseline: 1.4568x; 1.4568x over previous
import functools

import jax
import jax.numpy as jnp
from jax import lax
from jax.experimental import pallas as pl
from jax.experimental.pallas import tpu as pltpu
from jax.experimental.pallas import tpu_sc as plsc

N = 50000
H = 128
E = 320000

NC, NS = 2, 16
CH = 12544
NCHUNK = 2
NPAD = NC * NCHUNK * CH
STRIPE = CH // NS
DUMP = CH
EBLK = 128
SBLK = 2048
NSB = 10
ET = SBLK * NSB
EPAD = ET * NS
BM = 512


def _seg_sum_body(table, src_hbm, dst_hbm, zrow_hbm, zcnt_hbm, raw_hbm,
                  cnt_hbm, sbuf, dbuf, ldst, rows, ones_v, cbuf, wbc,
                  acc_sh, cnt_sh, gsem):
    c = lax.axis_index("c")
    s = lax.axis_index("s")
    pltpu.sync_copy(zcnt_hbm, cbuf)
    for j in range(EBLK // 16):
        ones_v[pl.ds(16 * j, 16)] = jnp.full((16,), 1.0, jnp.float32)

    for k in range(NCHUNK):
        base = (c * NCHUNK + k) * CH
        pltpu.sync_copy(zrow_hbm, rows)
        for q in range(STRIPE // EBLK):
            pltpu.sync_copy(rows, acc_sh.at[pl.ds(s * STRIPE + q * EBLK,
                                                  EBLK)])
        pltpu.sync_copy(rows.at[pl.ds(0, STRIPE % EBLK)],
                        acc_sh.at[pl.ds(s * STRIPE + STRIPE - STRIPE % EBLK,
                                        STRIPE % EBLK)])
        pltpu.sync_copy(cbuf, cnt_sh.at[pl.ds(s * STRIPE, STRIPE)])
        plsc.subcore_barrier()

        def sblk(t, _):
            pltpu.sync_copy(src_hbm.at[pl.ds(s * ET + t * SBLK, SBLK)], sbuf)
            pltpu.sync_copy(dst_hbm.at[pl.ds(s * ET + t * SBLK, SBLK)], dbuf)

            def blk(b, _):
                for j in range(EBLK // 16):
                    d = dbuf[pl.ds(b * EBLK + 16 * j, 16)]
                    l = d - base
                    m = (l >= 0) & (l < CH)
                    ldst[pl.ds(16 * j, 16)] = jnp.where(m, l, DUMP)
                pltpu.async_copy(table.at[sbuf.at[pl.ds(b * EBLK, EBLK)]],
                                 rows, gsem).wait()
                pltpu.sync_copy(rows, acc_sh.at[ldst], add=True)
                pltpu.sync_copy(ones_v, cnt_sh.at[ldst], add=True)
                return 0

            lax.fori_loop(0, SBLK // EBLK, blk, 0)
            return 0

        lax.fori_loop(0, NSB, sblk, 0)
        plsc.subcore_barrier()
        for q in range(STRIPE // EBLK):
            pltpu.sync_copy(acc_sh.at[pl.ds(s * STRIPE + q * EBLK, EBLK)],
                            rows)
            pltpu.sync_copy(rows,
                            raw_hbm.at[pl.ds(base + s * STRIPE + q * EBLK,
                                             EBLK)])
        rem = STRIPE % EBLK
        pltpu.sync_copy(acc_sh.at[pl.ds(s * STRIPE + STRIPE - rem, rem)],
                        rows.at[pl.ds(0, rem)])
        pltpu.sync_copy(rows.at[pl.ds(0, rem)],
                        raw_hbm.at[pl.ds(base + s * STRIPE + STRIPE - rem,
                                         rem)])
        pltpu.sync_copy(cnt_sh.at[pl.ds(s * STRIPE, STRIPE)], wbc)
        pltpu.sync_copy(wbc, cnt_hbm.at[pl.ds(base + s * STRIPE, STRIPE)])


_seg_sum = functools.partial(
    pl.kernel,
    out_type=[jax.ShapeDtypeStruct((NPAD, H), jnp.float32),
              jax.ShapeDtypeStruct((NPAD,), jnp.float32)],
    mesh=plsc.VectorSubcoreMesh(core_axis_name="c", subcore_axis_name="s",
                                num_cores=NC, num_subcores=NS),
    scratch_types=[
        pltpu.VMEM((SBLK,), jnp.int32),
        pltpu.VMEM((SBLK,), jnp.int32),
        pltpu.VMEM((EBLK,), jnp.int32),
        pltpu.VMEM((EBLK, H), jnp.float32),
        pltpu.VMEM((EBLK,), jnp.float32),
        pltpu.VMEM((STRIPE,), jnp.float32),
        pltpu.VMEM((STRIPE,), jnp.float32),
        pltpu.VMEM_SHARED((CH + 8, H), jnp.float32),
        pltpu.VMEM_SHARED((CH + 8,), jnp.float32),
        pltpu.SemaphoreType.DMA,
    ],
)(_seg_sum_body)


def _dense_body_sig(h_ref, raw_ref, ew_ref, cnt_ref, wself_ref, wmsg_ref,
                    b_ref, g_ref, beta_ref, o_ref):
    h = h_ref[...]
    scale = jax.nn.sigmoid(ew_ref[...]) / jnp.maximum(cnt_ref[...], 1.0)
    agg = jnp.dot(raw_ref[...], wmsg_ref[...],
                  preferred_element_type=jnp.float32) * scale
    pre = jnp.dot(h, wself_ref[...],
                  preferred_element_type=jnp.float32) + agg + b_ref[...]
    x = h + jnp.maximum(pre, 0.0)
    mu = jnp.mean(x, axis=-1, keepdims=True)
    var = jnp.mean((x - mu) ** 2, axis=-1, keepdims=True)
    o_ref[...] = (x - mu) * lax.rsqrt(var + 1e-5) * g_ref[...] + beta_ref[...]


def _dense_body_nosig(h_ref, raw_ref, cnt_ref, wself_ref, wmsg_ref,
                      b_ref, g_ref, beta_ref, o_ref):
    h = h_ref[...]
    scale = 1.0 / jnp.maximum(cnt_ref[...], 1.0)
    agg = jnp.dot(raw_ref[...], wmsg_ref[...],
                  preferred_element_type=jnp.float32) * scale
    pre = jnp.dot(h, wself_ref[...],
                  preferred_element_type=jnp.float32) + agg + b_ref[...]
    x = h + jnp.maximum(pre, 0.0)
    mu = jnp.mean(x, axis=-1, keepdims=True)
    var = jnp.mean((x - mu) ** 2, axis=-1, keepdims=True)
    o_ref[...] = (x - mu) * lax.rsqrt(var + 1e-5) * g_ref[...] + beta_ref[...]


_row_spec = pl.BlockSpec((BM, H), lambda i: (i, 0))
_col_spec = pl.BlockSpec((BM, 1), lambda i: (i, 0))
_w_spec = pl.BlockSpec((H, H), lambda i: (0, 0))
_vec_spec = pl.BlockSpec((1, H), lambda i: (0, 0))

_dense_sig = pl.pallas_call(
    _dense_body_sig,
    grid=(NPAD // BM,),
    in_specs=[_row_spec, _row_spec, _col_spec, _col_spec,
              _w_spec, _w_spec, _vec_spec, _vec_spec, _vec_spec],
    out_specs=_row_spec,
    out_shape=jax.ShapeDtypeStruct((N, H), jnp.float32),
)

_dense_nosig = pl.pallas_call(
    _dense_body_nosig,
    grid=(NPAD // BM,),
    in_specs=[_row_spec, _row_spec, _col_spec,
              _w_spec, _w_spec, _vec_spec, _vec_spec, _vec_spec],
    out_specs=_row_spec,
    out_shape=jax.ShapeDtypeStruct((N, H), jnp.float32),
)


def _pad_edges(src, dst):
    pad = EPAD - E
    srcp = jnp.concatenate([src.astype(jnp.int32),
                            jnp.zeros((pad,), jnp.int32)])
    dstp = jnp.concatenate([dst.astype(jnp.int32),
                            jnp.full((pad,), NPAD, jnp.int32)])
    return srcp, dstp


def _pad_rows(x):
    return jnp.concatenate([x, jnp.zeros((NPAD - N,) + x.shape[1:], x.dtype)])


def kernel(hD, hE, edge_d2e, edge_e2d, error_weights, W_d2e, W_e_self, b_e,
           g_e, beta_e, W_e2d, W_d_self, b_d, g_d, beta_d):
    hD2, hE2 = hD[0], hE[0]
    src1, dst1 = _pad_edges(edge_d2e[0], edge_d2e[1])
    src2, dst2 = _pad_edges(edge_e2d[0], edge_e2d[1])
    zrow = jnp.zeros((EBLK, H), jnp.float32)
    zcnt = jnp.zeros((STRIPE,), jnp.float32)

    hE_pad = _pad_rows(hE2)
    ew = _pad_rows(error_weights.reshape(N, 1))

    raw1, cnt1 = _seg_sum(hD2, src1, dst1, zrow, zcnt)
    hE_new = _dense_sig(hE_pad, raw1, ew, cnt1.reshape(NPAD, 1),
                        W_e_self.T, W_d2e.T, b_e.reshape(1, H),
                        g_e.reshape(1, H), beta_e.reshape(1, H))

    hD_pad = _pad_rows(hD2)
    raw2, cnt2 = _seg_sum(hE_new, src2, dst2, zrow, zcnt)
    hD_new = _dense_nosig(hD_pad, raw2, cnt2.reshape(NPAD, 1),
                          W_d_self.T, W_e2d.T, b_d.reshape(1, H),
                          g_d.reshape(1, H), beta_d.reshape(1, H))

    return (hD_new[None], hE_new[None])


# speedup vs baseline: 1.4711x; 1.0098x over previous
import functools

import jax
import jax.numpy as jnp
from jax import lax
from jax.experimental import pallas as pl
from jax.experimental.pallas import tpu as pltpu
from jax.experimental.pallas import tpu_sc as plsc

N = 50000
H = 128
E = 320000

NC, NS = 2, 16
CH = 12544
NCHUNK = 2
NPAD = NC * NCHUNK * CH
STRIPE = CH // NS
DUMP = CH
EBLK = 64
SBLK = 1024
NSB = 20
ET = SBLK * NSB
EPAD = ET * NS
NPAIR = SBLK // EBLK // 2
BM = 512


def _seg_sum_body(table, src_hbm, dst_hbm, zrow_hbm, zcnt_hbm, raw_hbm,
                  cnt_hbm, sbuf, dbuf, ldst0, ldst1, rows0, rows1, ones_v,
                  cbuf, wbc, acc_sh, cnt_sh, gsem0, gsem1, ssem0, ssem1):
    c = lax.axis_index("c")
    s = lax.axis_index("s")
    pltpu.sync_copy(zcnt_hbm, cbuf)
    for j in range(EBLK // 16):
        ones_v[pl.ds(16 * j, 16)] = jnp.full((16,), 1.0, jnp.float32)

    for k in range(NCHUNK):
        base = (c * NCHUNK + k) * CH
        pltpu.sync_copy(zrow_hbm, rows0)
        for q in range(STRIPE // EBLK):
            pltpu.sync_copy(rows0, acc_sh.at[pl.ds(s * STRIPE + q * EBLK,
                                                   EBLK)])
        rem = STRIPE % EBLK
        pltpu.sync_copy(rows0.at[pl.ds(0, rem)],
                        acc_sh.at[pl.ds(s * STRIPE + STRIPE - rem, rem)])
        pltpu.sync_copy(cbuf, cnt_sh.at[pl.ds(s * STRIPE, STRIPE)])
        plsc.subcore_barrier()

        def sblk(t, _):
            pltpu.sync_copy(src_hbm.at[pl.ds(s * ET + t * SBLK, SBLK)], sbuf)
            pltpu.sync_copy(dst_hbm.at[pl.ds(s * ET + t * SBLK, SBLK)], dbuf)

            def pair(p, _):
                bA, bB = 2 * p, 2 * p + 1

                def launch(b, ldst, rows, gsem):
                    for j in range(EBLK // 16):
                        d = dbuf[pl.ds(b * EBLK + 16 * j, 16)]
                        l = d - base
                        m = (l >= 0) & (l < CH)
                        ldst[pl.ds(16 * j, 16)] = jnp.where(m, l, DUMP)
                    pltpu.async_copy(
                        table.at[sbuf.at[pl.ds(b * EBLK, EBLK)]], rows, gsem)

                def scatter(b, ldst, rows, gsem, ssem):
                    pltpu.make_async_copy(
                        table.at[sbuf.at[pl.ds(b * EBLK, EBLK)]], rows,
                        gsem).wait()
                    pltpu.async_copy(rows, acc_sh.at[ldst], ssem, add=True)
                    pltpu.async_copy(ones_v, cnt_sh.at[ldst], ssem, add=True)

                def drain(ldst, rows, ssem):
                    pltpu.make_async_copy(rows, acc_sh.at[ldst], ssem).wait()
                    pltpu.make_async_copy(ones_v, cnt_sh.at[ldst],
                                          ssem).wait()

                launch(bA, ldst0, rows0, gsem0)
                launch(bB, ldst1, rows1, gsem1)
                scatter(bA, ldst0, rows0, gsem0, ssem0)
                scatter(bB, ldst1, rows1, gsem1, ssem1)
                drain(ldst0, rows0, ssem0)
                drain(ldst1, rows1, ssem1)
                return 0

            lax.fori_loop(0, NPAIR, pair, 0)
            return 0

        lax.fori_loop(0, NSB, sblk, 0)
        plsc.subcore_barrier()
        for q in range(STRIPE // EBLK):
            pltpu.sync_copy(acc_sh.at[pl.ds(s * STRIPE + q * EBLK, EBLK)],
                            rows0)
            pltpu.sync_copy(rows0,
                            raw_hbm.at[pl.ds(base + s * STRIPE + q * EBLK,
                                             EBLK)])
        pltpu.sync_copy(acc_sh.at[pl.ds(s * STRIPE + STRIPE - rem, rem)],
                        rows0.at[pl.ds(0, rem)])
        pltpu.sync_copy(rows0.at[pl.ds(0, rem)],
                        raw_hbm.at[pl.ds(base + s * STRIPE + STRIPE - rem,
                                         rem)])
        pltpu.sync_copy(cnt_sh.at[pl.ds(s * STRIPE, STRIPE)], wbc)
        pltpu.sync_copy(wbc, cnt_hbm.at[pl.ds(base + s * STRIPE, STRIPE)])


_seg_sum = functools.partial(
    pl.kernel,
    out_type=[jax.ShapeDtypeStruct((NPAD, H), jnp.float32),
              jax.ShapeDtypeStruct((NPAD,), jnp.float32)],
    mesh=plsc.VectorSubcoreMesh(core_axis_name="c", subcore_axis_name="s",
                                num_cores=NC, num_subcores=NS),
    scratch_types=[
        pltpu.VMEM((SBLK,), jnp.int32),
        pltpu.VMEM((SBLK,), jnp.int32),
        pltpu.VMEM((EBLK,), jnp.int32),
        pltpu.VMEM((EBLK,), jnp.int32),
        pltpu.VMEM((EBLK, H), jnp.float32),
        pltpu.VMEM((EBLK, H), jnp.float32),
        pltpu.VMEM((EBLK,), jnp.float32),
        pltpu.VMEM((STRIPE,), jnp.float32),
        pltpu.VMEM((STRIPE,), jnp.float32),
        pltpu.VMEM_SHARED((CH + 8, H), jnp.float32),
        pltpu.VMEM_SHARED((CH + 8,), jnp.float32),
        pltpu.SemaphoreType.DMA,
        pltpu.SemaphoreType.DMA,
        pltpu.SemaphoreType.DMA,
        pltpu.SemaphoreType.DMA,
    ],
)(_seg_sum_body)


def _dense_body_sig(h_ref, raw_ref, ew_ref, cnt_ref, wself_ref, wmsg_ref,
                    b_ref, g_ref, beta_ref, o_ref):
    h = h_ref[...]
    scale = jax.nn.sigmoid(ew_ref[...]) / jnp.maximum(cnt_ref[...], 1.0)
    agg = jnp.dot(raw_ref[...], wmsg_ref[...],
                  preferred_element_type=jnp.float32) * scale
    pre = jnp.dot(h, wself_ref[...],
                  preferred_element_type=jnp.float32) + agg + b_ref[...]
    x = h + jnp.maximum(pre, 0.0)
    mu = jnp.mean(x, axis=-1, keepdims=True)
    var = jnp.mean((x - mu) ** 2, axis=-1, keepdims=True)
    o_ref[...] = (x - mu) * lax.rsqrt(var + 1e-5) * g_ref[...] + beta_ref[...]


def _dense_body_nosig(h_ref, raw_ref, cnt_ref, wself_ref, wmsg_ref,
                      b_ref, g_ref, beta_ref, o_ref):
    h = h_ref[...]
    scale = 1.0 / jnp.maximum(cnt_ref[...], 1.0)
    agg = jnp.dot(raw_ref[...], wmsg_ref[...],
                  preferred_element_type=jnp.float32) * scale
    pre = jnp.dot(h, wself_ref[...],
                  preferred_element_type=jnp.float32) + agg + b_ref[...]
    x = h + jnp.maximum(pre, 0.0)
    mu = jnp.mean(x, axis=-1, keepdims=True)
    var = jnp.mean((x - mu) ** 2, axis=-1, keepdims=True)
    o_ref[...] = (x - mu) * lax.rsqrt(var + 1e-5) * g_ref[...] + beta_ref[...]


_row_spec = pl.BlockSpec((BM, H), lambda i: (i, 0))
_col_spec = pl.BlockSpec((BM, 1), lambda i: (i, 0))
_w_spec = pl.BlockSpec((H, H), lambda i: (0, 0))
_vec_spec = pl.BlockSpec((1, H), lambda i: (0, 0))

_dense_sig = pl.pallas_call(
    _dense_body_sig,
    grid=(NPAD // BM,),
    in_specs=[_row_spec, _row_spec, _col_spec, _col_spec,
              _w_spec, _w_spec, _vec_spec, _vec_spec, _vec_spec],
    out_specs=_row_spec,
    out_shape=jax.ShapeDtypeStruct((N, H), jnp.float32),
)

_dense_nosig = pl.pallas_call(
    _dense_body_nosig,
    grid=(NPAD // BM,),
    in_specs=[_row_spec, _row_spec, _col_spec,
              _w_spec, _w_spec, _vec_spec, _vec_spec, _vec_spec],
    out_specs=_row_spec,
    out_shape=jax.ShapeDtypeStruct((N, H), jnp.float32),
)


def _pad_edges(src, dst):
    pad = EPAD - E
    srcp = jnp.concatenate([src.astype(jnp.int32),
                            jnp.zeros((pad,), jnp.int32)])
    dstp = jnp.concatenate([dst.astype(jnp.int32),
                            jnp.full((pad,), NPAD, jnp.int32)])
    return srcp, dstp


def _pad_rows(x):
    return jnp.concatenate([x, jnp.zeros((NPAD - N,) + x.shape[1:], x.dtype)])


def kernel(hD, hE, edge_d2e, edge_e2d, error_weights, W_d2e, W_e_self, b_e,
           g_e, beta_e, W_e2d, W_d_self, b_d, g_d, beta_d):
    hD2, hE2 = hD[0], hE[0]
    src1, dst1 = _pad_edges(edge_d2e[0], edge_d2e[1])
    src2, dst2 = _pad_edges(edge_e2d[0], edge_e2d[1])
    zrow = jnp.zeros((EBLK, H), jnp.float32)
    zcnt = jnp.zeros((STRIPE,), jnp.float32)

    hE_pad = _pad_rows(hE2)
    ew = _pad_rows(error_weights.reshape(N, 1))

    raw1, cnt1 = _seg_sum(hD2, src1, dst1, zrow, zcnt)
    hE_new = _dense_sig(hE_pad, raw1, ew, cnt1.reshape(NPAD, 1),
                        W_e_self.T, W_d2e.T, b_e.reshape(1, H),
                        g_e.reshape(1, H), beta_e.reshape(1, H))

    hD_pad = _pad_rows(hD2)
    raw2, cnt2 = _seg_sum(hE_new, src2, dst2, zrow, zcnt)
    hD_new = _dense_nosig(hD_pad, raw2, cnt2.reshape(NPAD, 1),
                          W_d_self.T, W_e2d.T, b_d.reshape(1, H),
                          g_d.reshape(1, H), beta_d.reshape(1, H))

    return (hD_new[None], hE_new[None])
